# baseline (speedup 1.0000x reference)
import functools

import jax
import jax.numpy as jnp
from jax import lax
from jax.experimental import pallas as pl
from jax.experimental.pallas import tpu as pltpu
from jax.experimental.pallas import tpu_sc as plsc

_N = 10000
_E = 320000
_F = 128
_FH = 64
_CLS = 16
_BN_EPS = 1e-5

_NC = 2
_NS = 16
_NW = _NC * _NS
_NP = 10240
_RPT = _NP // _NS
_EPT = _E // _NW
_K = 80
_NCHUNK = 250
_EPAD = _NS * _NCHUNK * _K
_NBUF = 5

def _mesh():
    return plsc.VectorSubcoreMesh(
        core_axis_name="c", subcore_axis_name="s",
        num_cores=_NC, num_subcores=_NS)


@functools.cache
def _build_deg_call():
    return pl.kernel(
        _deg_body,
        out_type=jax.ShapeDtypeStruct((_NC, 1, _NP), jnp.float32),
        mesh=_mesh(),
        compiler_params=pltpu.CompilerParams(
            needs_layout_passes=False, use_tc_tiling_on_sc=False),
        scratch_types=[
            pltpu.VMEM((1, _EPT), jnp.int32),
            pltpu.VMEM((_NP,), jnp.float32),
            pltpu.VMEM((_RPT,), jnp.float32),
            pltpu.VMEM((_RPT,), jnp.float32),
            pltpu.VMEM_SHARED((_NS, _NP), jnp.float32),
        ],
    )


def _deg_body(dst_hbm, out_hbm, dst_v, hist_v, acc_v, tmp_v, part_sp):
    c = lax.axis_index("c")
    s = lax.axis_index("s")
    w = c * _NS + s
    pltpu.sync_copy(dst_hbm.at[w], dst_v)

    @pl.loop(0, _NP // 16, unroll=8)
    def _(i):
        hist_v[pl.ds(i * 16, 16)] = jnp.zeros((16,), jnp.float32)

    ones = jnp.full((16,), 1.0, jnp.float32)

    @pl.loop(0, _EPT // 16, unroll=8)
    def _(i):
        idx = dst_v[0, pl.ds(i * 16, 16)]
        plsc.addupdate_scatter(hist_v, [idx], ones)

    pltpu.sync_copy(hist_v, part_sp.at[s])
    plsc.subcore_barrier()

    pltpu.sync_copy(part_sp.at[0, pl.ds(s * _RPT, _RPT)], acc_v)
    for r in range(1, _NS):
        pltpu.sync_copy(part_sp.at[r, pl.ds(s * _RPT, _RPT)], tmp_v)

        @pl.loop(0, _RPT // 16)
        def _(k):
            sl = pl.ds(k * 16, 16)
            acc_v[sl] = acc_v[sl] + tmp_v[sl]

    pltpu.sync_copy(acc_v, out_hbm.at[c, 0, pl.ds(s * _RPT, _RPT)])


@functools.cache
def _build_agg_call():
    return pl.kernel(
        _agg_body,
        out_type=jax.ShapeDtypeStruct((_NC, _NP, _FH), jnp.float32),
        mesh=_mesh(),
        compiler_params=pltpu.CompilerParams(
            needs_layout_passes=False, use_tc_tiling_on_sc=False),
        scratch_types=[
            pltpu.VMEM((_NCHUNK, _K), jnp.int32),
            pltpu.VMEM((_NCHUNK, _K), jnp.int32),
            pltpu.VMEM((_NBUF, _K, _FH), jnp.float32),
            pltpu.VMEM_SHARED((_NP, _FH), jnp.float32),
            pltpu.SemaphoreType.DMA((_NBUF,)),
            pltpu.SemaphoreType.DMA((_NBUF,)),
        ],
    )


def _agg_body(th_hbm, src2_hbm, dstp_hbm, z_hbm, out_hbm, src_v, dst_v,
              rows_v, agg_sp, gsem, ssem):
    c = lax.axis_index("c")
    s = lax.axis_index("s")
    pltpu.sync_copy(src2_hbm.at[c, s], src_v)
    pltpu.sync_copy(dstp_hbm.at[s], dst_v)
    pltpu.sync_copy(z_hbm, agg_sp.at[pl.ds(s * _RPT, _RPT)])
    plsc.subcore_barrier()

    for b in range(_NBUF):
        pltpu.async_copy(th_hbm.at[src_v.at[b]], rows_v.at[b], gsem.at[b])

    @pl.loop(0, _NCHUNK, step=_NBUF)
    def _(g):
        for b in range(_NBUF):
            j = g + b
            pltpu.make_async_copy(
                th_hbm.at[src_v.at[j]], rows_v.at[b], gsem.at[b]).wait()
            pltpu.async_copy(
                rows_v.at[b], agg_sp.at[dst_v.at[j]], ssem.at[b], add=True)
            jn = j + _NBUF

            @pl.when(jn < _NCHUNK)
            def _():
                pltpu.make_async_copy(
                    rows_v.at[b], agg_sp.at[dst_v.at[j]], ssem.at[b]).wait()
                pltpu.async_copy(
                    th_hbm.at[src_v.at[jn]], rows_v.at[b], gsem.at[b])

    for b in range(_NBUF):
        jl = _NCHUNK - _NBUF + b
        pltpu.make_async_copy(
            rows_v.at[b], agg_sp.at[dst_v.at[jl]], ssem.at[b]).wait()
    plsc.subcore_barrier()
    pltpu.sync_copy(agg_sp.at[pl.ds(s * _RPT, _RPT)],
                    out_hbm.at[c, pl.ds(s * _RPT, _RPT)])


def _agg_sum(p_ref):
    return jnp.concatenate([p_ref[0, :_N, :], p_ref[1, :_N, :]], axis=1)


def _tc_a_body(seq_ref, w1_ref, dinv_ref, out_ref):
    h = jnp.dot(seq_ref[...], w1_ref[...], preferred_element_type=jnp.float32)
    out_ref[...] = h * dinv_ref[...]


_tc_a = pl.pallas_call(
    _tc_a_body, out_shape=jax.ShapeDtypeStruct((_N, _F), jnp.float32))


def _tc_b_body(p_ref, h1p_ref, dinv_ref, b1_ref, bnw_ref, bnb_ref, w2_ref,
               out_ref):
    res1 = (_agg_sum(p_ref) + h1p_ref[...]) * dinv_ref[...] + b1_ref[...]
    mean = jnp.mean(res1, axis=0, keepdims=True)
    var = jnp.mean(res1 * res1, axis=0, keepdims=True) - mean * mean
    x1 = (res1 - mean) * jax.lax.rsqrt(var + _BN_EPS) * bnw_ref[...] + bnb_ref[...]
    x1 = jnp.maximum(x1, 0.0)
    out_ref[...] = jnp.dot(
        x1, w2_ref[...], preferred_element_type=jnp.float32) * dinv_ref[...]


_tc_b = pl.pallas_call(
    _tc_b_body, out_shape=jax.ShapeDtypeStruct((_N, _F), jnp.float32))


def _tc_c_body(p_ref, h2p_ref, dinv_ref, b2_ref, wc_ref, bc_ref, out_ref):
    res2 = (_agg_sum(p_ref) + h2p_ref[...]) * dinv_ref[...] + b2_ref[...]
    out_ref[...] = jnp.dot(
        res2, wc_ref[...], preferred_element_type=jnp.float32) + bc_ref[...]


_tc_c = pl.pallas_call(
    _tc_c_body, out_shape=jax.ShapeDtypeStruct((_N, _CLS), jnp.float32))


def kernel(seq, edge_index, W1, b1, bn_w, bn_b, W2, b2, Wc, bc):
    ei = edge_index.astype(jnp.int32)
    src_x2 = 2 * ei[0]
    src2 = jnp.stack([src_x2, src_x2 + 1]).reshape(_NC, _NS, _NCHUNK, _K)
    dst3 = ei[1].reshape(_NS, _NCHUNK, _K)
    dstf3 = ei[1].reshape(_NW, 1, _EPT)
    zrows = jnp.zeros((_RPT, _FH), jnp.float32)

    degp = _build_deg_call()(dstf3)
    dinv = jax.lax.rsqrt(1.0 + degp[0, 0, :_N] + degp[1, 0, :_N])[:, None]

    h1p = _tc_a(seq, W1, dinv)
    p1 = _build_agg_call()(h1p.reshape(2 * _N, _FH), src2, dst3, zrows)
    h2p = _tc_b(p1, h1p, dinv, b1.reshape(1, _F), bn_w.reshape(1, _F),
                bn_b.reshape(1, _F), W2)
    p2 = _build_agg_call()(h2p.reshape(2 * _N, _FH), src2, dst3, zrows)
    out = _tc_c(p2, h2p, dinv, b2.reshape(1, _F), Wc, bc.reshape(1, _CLS))
    return out


# speedup vs baseline: 1.0122x; 1.0122x over previous
import functools

import jax
import jax.numpy as jnp
from jax import lax
from jax.experimental import pallas as pl
from jax.experimental.pallas import tpu as pltpu
from jax.experimental.pallas import tpu_sc as plsc

_N = 10000
_E = 320000
_F = 128
_FH = 64
_CLS = 16
_BN_EPS = 1e-5

_NC = 2
_NS = 16
_NW = _NC * _NS
_NP = 10240
_RPT = _NP // _NS
_EPT = _E // _NW
_K = 80
_NCHUNK = 250
_EPAD = _NS * _NCHUNK * _K
_NBUF = 5

def _mesh():
    return plsc.VectorSubcoreMesh(
        core_axis_name="c", subcore_axis_name="s",
        num_cores=_NC, num_subcores=_NS)


@functools.cache
def _build_deg_call():
    return pl.kernel(
        _deg_body,
        out_type=jax.ShapeDtypeStruct((_NC, 1, _NP), jnp.float32),
        mesh=_mesh(),
        compiler_params=pltpu.CompilerParams(
            needs_layout_passes=False, use_tc_tiling_on_sc=False),
        scratch_types=[
            pltpu.VMEM((1, _EPT), jnp.int32),
            pltpu.VMEM((_NP,), jnp.float32),
            pltpu.VMEM((_RPT,), jnp.float32),
            pltpu.VMEM((_RPT,), jnp.float32),
            pltpu.VMEM_SHARED((_NS, _NP), jnp.float32),
        ],
    )


def _deg_body(dst_hbm, out_hbm, dst_v, hist_v, acc_v, tmp_v, part_sp):
    c = lax.axis_index("c")
    s = lax.axis_index("s")
    w = c * _NS + s
    pltpu.sync_copy(dst_hbm.at[w], dst_v)

    @pl.loop(0, _NP // 16, unroll=8)
    def _(i):
        hist_v[pl.ds(i * 16, 16)] = jnp.zeros((16,), jnp.float32)

    ones = jnp.full((16,), 1.0, jnp.float32)

    @pl.loop(0, _EPT // 16, unroll=8)
    def _(i):
        idx = dst_v[0, pl.ds(i * 16, 16)]
        plsc.addupdate_scatter(hist_v, [idx], ones)

    pltpu.sync_copy(hist_v, part_sp.at[s])
    plsc.subcore_barrier()

    pltpu.sync_copy(part_sp.at[0, pl.ds(s * _RPT, _RPT)], acc_v)
    for r in range(1, _NS):
        pltpu.sync_copy(part_sp.at[r, pl.ds(s * _RPT, _RPT)], tmp_v)

        @pl.loop(0, _RPT // 16)
        def _(k):
            sl = pl.ds(k * 16, 16)
            acc_v[sl] = acc_v[sl] + tmp_v[sl]

    pltpu.sync_copy(acc_v, out_hbm.at[c, 0, pl.ds(s * _RPT, _RPT)])


@functools.cache
def _build_agg_call():
    return pl.kernel(
        _agg_body,
        out_type=jax.ShapeDtypeStruct((_NC, _NP, _FH), jnp.float32),
        mesh=_mesh(),
        compiler_params=pltpu.CompilerParams(
            needs_layout_passes=False, use_tc_tiling_on_sc=False),
        scratch_types=[
            pltpu.VMEM((_NCHUNK, _K), jnp.int32),
            pltpu.VMEM((_NCHUNK, _K), jnp.int32),
            pltpu.VMEM((_NBUF, _K, _FH), jnp.float32),
            pltpu.VMEM_SHARED((_NP, _FH), jnp.float32),
            pltpu.SemaphoreType.DMA((_NBUF,)),
            pltpu.SemaphoreType.DMA((_NBUF,)),
        ],
    )


def _agg_body(th_hbm, src2_hbm, dstp_hbm, z_hbm, out_hbm, src_v, dst_v,
              rows_v, agg_sp, gsem, ssem):
    c = lax.axis_index("c")
    s = lax.axis_index("s")
    cp_s = pltpu.async_copy(src2_hbm.at[c, s], src_v, gsem.at[0])
    cp_d = pltpu.async_copy(dstp_hbm.at[s], dst_v, gsem.at[1])
    cp_z = pltpu.async_copy(z_hbm, agg_sp.at[pl.ds(s * _RPT, _RPT)],
                            gsem.at[2])
    cp_s.wait()
    cp_d.wait()
    cp_z.wait()
    plsc.subcore_barrier()

    for b in range(_NBUF):
        pltpu.async_copy(th_hbm.at[src_v.at[b]], rows_v.at[b], gsem.at[b])

    @pl.loop(0, _NCHUNK, step=_NBUF)
    def _(g):
        for b in range(_NBUF):
            j = g + b
            pltpu.make_async_copy(
                th_hbm.at[src_v.at[j]], rows_v.at[b], gsem.at[b]).wait()
            pltpu.async_copy(
                rows_v.at[b], agg_sp.at[dst_v.at[j]], ssem.at[b], add=True)
            jn = j + _NBUF

            @pl.when(jn < _NCHUNK)
            def _():
                pltpu.make_async_copy(
                    rows_v.at[b], agg_sp.at[dst_v.at[j]], ssem.at[b]).wait()
                pltpu.async_copy(
                    th_hbm.at[src_v.at[jn]], rows_v.at[b], gsem.at[b])

    for b in range(_NBUF):
        jl = _NCHUNK - _NBUF + b
        pltpu.make_async_copy(
            rows_v.at[b], agg_sp.at[dst_v.at[jl]], ssem.at[b]).wait()
    plsc.subcore_barrier()
    pltpu.sync_copy(agg_sp.at[pl.ds(s * _RPT, _RPT)],
                    out_hbm.at[c, pl.ds(s * _RPT, _RPT)])


def _agg_sum(p_ref):
    return jnp.concatenate([p_ref[0, :_N, :], p_ref[1, :_N, :]], axis=1)


def _tc_a_body(seq_ref, w1_ref, dinv_ref, out_ref):
    h = jnp.dot(seq_ref[...], w1_ref[...], preferred_element_type=jnp.float32)
    out_ref[...] = h * dinv_ref[...]


_tc_a = pl.pallas_call(
    _tc_a_body, out_shape=jax.ShapeDtypeStruct((_N, _F), jnp.float32))


def _tc_b_body(p_ref, h1p_ref, dinv_ref, b1_ref, bnw_ref, bnb_ref, w2_ref,
               out_ref):
    res1 = (_agg_sum(p_ref) + h1p_ref[...]) * dinv_ref[...] + b1_ref[...]
    mean = jnp.mean(res1, axis=0, keepdims=True)
    var = jnp.mean(res1 * res1, axis=0, keepdims=True) - mean * mean
    x1 = (res1 - mean) * jax.lax.rsqrt(var + _BN_EPS) * bnw_ref[...] + bnb_ref[...]
    x1 = jnp.maximum(x1, 0.0)
    out_ref[...] = jnp.dot(
        x1, w2_ref[...], preferred_element_type=jnp.float32) * dinv_ref[...]


_tc_b = pl.pallas_call(
    _tc_b_body, out_shape=jax.ShapeDtypeStruct((_N, _F), jnp.float32))


def _tc_c_body(p_ref, h2p_ref, dinv_ref, b2_ref, wc_ref, bc_ref, out_ref):
    res2 = (_agg_sum(p_ref) + h2p_ref[...]) * dinv_ref[...] + b2_ref[...]
    out_ref[...] = jnp.dot(
        res2, wc_ref[...], preferred_element_type=jnp.float32) + bc_ref[...]


_tc_c = pl.pallas_call(
    _tc_c_body, out_shape=jax.ShapeDtypeStruct((_N, _CLS), jnp.float32))


def kernel(seq, edge_index, W1, b1, bn_w, bn_b, W2, b2, Wc, bc):
    ei = edge_index.astype(jnp.int32)
    src_x2 = 2 * ei[0]
    src2 = jnp.stack([src_x2, src_x2 + 1]).reshape(_NC, _NS, _NCHUNK, _K)
    dst3 = ei[1].reshape(_NS, _NCHUNK, _K)
    dstf3 = ei[1].reshape(_NW, 1, _EPT)
    zrows = jnp.zeros((_RPT, _FH), jnp.float32)

    degp = _build_deg_call()(dstf3)
    dinv = jax.lax.rsqrt(1.0 + degp[0, 0, :_N] + degp[1, 0, :_N])[:, None]

    h1p = _tc_a(seq, W1, dinv)
    p1 = _build_agg_call()(h1p.reshape(2 * _N, _FH), src2, dst3, zrows)
    h2p = _tc_b(p1, h1p, dinv, b1.reshape(1, _F), bn_w.reshape(1, _F),
                bn_b.reshape(1, _F), W2)
    p2 = _build_agg_call()(h2p.reshape(2 * _N, _FH), src2, dst3, zrows)
    out = _tc_c(p2, h2p, dinv, b2.reshape(1, _F), Wc, bc.reshape(1, _CLS))
    return out


# speedup vs baseline: 1.0151x; 1.0029x over previous
import functools

import jax
import jax.numpy as jnp
from jax import lax
from jax.experimental import pallas as pl
from jax.experimental.pallas import tpu as pltpu
from jax.experimental.pallas import tpu_sc as plsc

_N = 10000
_E = 320000
_F = 128
_FH = 64
_CLS = 16
_BN_EPS = 1e-5

_NC = 2
_NS = 16
_NW = _NC * _NS
_NP = 10240
_RPT = _NP // _NS
_EPT = _E // _NW
_K = 80
_NCHUNK = 250
_EPAD = _NS * _NCHUNK * _K
_NBUF = 5

def _mesh():
    return plsc.VectorSubcoreMesh(
        core_axis_name="c", subcore_axis_name="s",
        num_cores=_NC, num_subcores=_NS)


@functools.cache
def _build_deg_call():
    return pl.kernel(
        _deg_body,
        out_type=jax.ShapeDtypeStruct((_NC, 1, _NP), jnp.float32),
        mesh=_mesh(),
        compiler_params=pltpu.CompilerParams(
            needs_layout_passes=False, use_tc_tiling_on_sc=False),
        scratch_types=[
            pltpu.VMEM((1, _EPT), jnp.int32),
            pltpu.VMEM((_NP,), jnp.float32),
            pltpu.VMEM((_RPT,), jnp.float32),
            pltpu.VMEM((2, _RPT), jnp.float32),
            pltpu.VMEM_SHARED((_NS, _NP), jnp.float32),
            pltpu.SemaphoreType.DMA((2,)),
        ],
    )


def _deg_body(dst_hbm, out_hbm, dst_v, hist_v, acc_v, tmp_v, part_sp, sem):
    c = lax.axis_index("c")
    s = lax.axis_index("s")
    w = c * _NS + s
    cp_idx = pltpu.async_copy(dst_hbm.at[w], dst_v, sem.at[0])

    @pl.loop(0, _NP // 16, unroll=8)
    def _(i):
        hist_v[pl.ds(i * 16, 16)] = jnp.zeros((16,), jnp.float32)

    cp_idx.wait()
    ones = jnp.full((16,), 1.0, jnp.float32)

    @pl.loop(0, _EPT // 16, unroll=8)
    def _(i):
        idx = dst_v[0, pl.ds(i * 16, 16)]
        plsc.addupdate_scatter(hist_v, [idx], ones)

    pltpu.sync_copy(hist_v, part_sp.at[s])
    plsc.subcore_barrier()

    col = pl.ds(s * _RPT, _RPT)
    pltpu.sync_copy(part_sp.at[0, col], acc_v)
    pltpu.async_copy(part_sp.at[1, col], tmp_v.at[1], sem.at[1])
    for r in range(1, _NS):
        b = r % 2
        pltpu.make_async_copy(part_sp.at[r, col], tmp_v.at[b],
                              sem.at[b]).wait()
        if r + 1 < _NS:
            pltpu.async_copy(part_sp.at[r + 1, col], tmp_v.at[1 - b],
                             sem.at[1 - b])

        @pl.loop(0, _RPT // 16, unroll=4)
        def _(k):
            sl = pl.ds(k * 16, 16)
            acc_v[sl] = acc_v[sl] + tmp_v[b, sl]

    pltpu.sync_copy(acc_v, out_hbm.at[c, 0, pl.ds(s * _RPT, _RPT)])


@functools.cache
def _build_agg_call():
    return pl.kernel(
        _agg_body,
        out_type=jax.ShapeDtypeStruct((_NC, _NP, _FH), jnp.float32),
        mesh=_mesh(),
        compiler_params=pltpu.CompilerParams(
            needs_layout_passes=False, use_tc_tiling_on_sc=False),
        scratch_types=[
            pltpu.VMEM((_NCHUNK, _K), jnp.int32),
            pltpu.VMEM((_NCHUNK, _K), jnp.int32),
            pltpu.VMEM((_NBUF, _K, _FH), jnp.float32),
            pltpu.VMEM_SHARED((_NP, _FH), jnp.float32),
            pltpu.SemaphoreType.DMA((_NBUF,)),
            pltpu.SemaphoreType.DMA((_NBUF,)),
        ],
    )


def _agg_body(th_hbm, src2_hbm, dstp_hbm, z_hbm, out_hbm, src_v, dst_v,
              rows_v, agg_sp, gsem, ssem):
    c = lax.axis_index("c")
    s = lax.axis_index("s")
    cp_s = pltpu.async_copy(src2_hbm.at[c, s], src_v, gsem.at[0])
    cp_d = pltpu.async_copy(dstp_hbm.at[s], dst_v, gsem.at[1])
    cp_z = pltpu.async_copy(z_hbm, agg_sp.at[pl.ds(s * _RPT, _RPT)],
                            gsem.at[2])
    cp_s.wait()
    cp_d.wait()
    cp_z.wait()
    plsc.subcore_barrier()

    for b in range(_NBUF):
        pltpu.async_copy(th_hbm.at[src_v.at[b]], rows_v.at[b], gsem.at[b])

    @pl.loop(0, _NCHUNK, step=_NBUF)
    def _(g):
        for b in range(_NBUF):
            j = g + b
            pltpu.make_async_copy(
                th_hbm.at[src_v.at[j]], rows_v.at[b], gsem.at[b]).wait()
            pltpu.async_copy(
                rows_v.at[b], agg_sp.at[dst_v.at[j]], ssem.at[b], add=True)
            jn = j + _NBUF

            @pl.when(jn < _NCHUNK)
            def _():
                pltpu.make_async_copy(
                    rows_v.at[b], agg_sp.at[dst_v.at[j]], ssem.at[b]).wait()
                pltpu.async_copy(
                    th_hbm.at[src_v.at[jn]], rows_v.at[b], gsem.at[b])

    for b in range(_NBUF):
        jl = _NCHUNK - _NBUF + b
        pltpu.make_async_copy(
            rows_v.at[b], agg_sp.at[dst_v.at[jl]], ssem.at[b]).wait()
    plsc.subcore_barrier()
    pltpu.sync_copy(agg_sp.at[pl.ds(s * _RPT, _RPT)],
                    out_hbm.at[c, pl.ds(s * _RPT, _RPT)])


def _agg_sum(p_ref):
    return jnp.concatenate([p_ref[0, :_N, :], p_ref[1, :_N, :]], axis=1)


def _tc_a_body(seq_ref, w1_ref, dinv_ref, out_ref):
    h = jnp.dot(seq_ref[...], w1_ref[...], preferred_element_type=jnp.float32)
    out_ref[...] = h * dinv_ref[...]


_tc_a = pl.pallas_call(
    _tc_a_body, out_shape=jax.ShapeDtypeStruct((_N, _F), jnp.float32))


def _tc_b_body(p_ref, h1p_ref, dinv_ref, b1_ref, bnw_ref, bnb_ref, w2_ref,
               out_ref):
    res1 = (_agg_sum(p_ref) + h1p_ref[...]) * dinv_ref[...] + b1_ref[...]
    mean = jnp.mean(res1, axis=0, keepdims=True)
    var = jnp.mean(res1 * res1, axis=0, keepdims=True) - mean * mean
    x1 = (res1 - mean) * jax.lax.rsqrt(var + _BN_EPS) * bnw_ref[...] + bnb_ref[...]
    x1 = jnp.maximum(x1, 0.0)
    out_ref[...] = jnp.dot(
        x1, w2_ref[...], preferred_element_type=jnp.float32) * dinv_ref[...]


_tc_b = pl.pallas_call(
    _tc_b_body, out_shape=jax.ShapeDtypeStruct((_N, _F), jnp.float32))


def _tc_c_body(p_ref, h2p_ref, dinv_ref, b2_ref, wc_ref, bc_ref, out_ref):
    res2 = (_agg_sum(p_ref) + h2p_ref[...]) * dinv_ref[...] + b2_ref[...]
    out_ref[...] = jnp.dot(
        res2, wc_ref[...], preferred_element_type=jnp.float32) + bc_ref[...]


_tc_c = pl.pallas_call(
    _tc_c_body, out_shape=jax.ShapeDtypeStruct((_N, _CLS), jnp.float32))


def kernel(seq, edge_index, W1, b1, bn_w, bn_b, W2, b2, Wc, bc):
    ei = edge_index.astype(jnp.int32)
    src_x2 = 2 * ei[0]
    src2 = jnp.stack([src_x2, src_x2 + 1]).reshape(_NC, _NS, _NCHUNK, _K)
    dst3 = ei[1].reshape(_NS, _NCHUNK, _K)
    dstf3 = ei[1].reshape(_NW, 1, _EPT)
    zrows = jnp.zeros((_RPT, _FH), jnp.float32)

    degp = _build_deg_call()(dstf3)
    dinv = jax.lax.rsqrt(1.0 + degp[0, 0, :_N] + degp[1, 0, :_N])[:, None]

    h1p = _tc_a(seq, W1, dinv)
    p1 = _build_agg_call()(h1p.reshape(2 * _N, _FH), src2, dst3, zrows)
    h2p = _tc_b(p1, h1p, dinv, b1.reshape(1, _F), bn_w.reshape(1, _F),
                bn_b.reshape(1, _F), W2)
    p2 = _build_agg_call()(h2p.reshape(2 * _N, _FH), src2, dst3, zrows)
    out = _tc_c(p2, h2p, dinv, b2.reshape(1, _F), Wc, bc.reshape(1, _CLS))
    return out
